# speedup vs baseline: 1.4408x; 1.4408x over previous
import jax
import jax.numpy as jnp
from jax import lax
from jax.experimental import pallas as pl
from jax.experimental.pallas import tpu as pltpu
from jax.experimental.pallas import tpu_sc as plsc

BATCH = 4
SEQ_LEN = 4096
D_MODEL = 1024
TOT = BATCH * SEQ_LEN

NUM_CORES = 2
NUM_SUBCORES = 16
NW = NUM_CORES * NUM_SUBCORES
B_PER_W = TOT // NW

CHUNK = 32
NCHUNK = B_PER_W // CHUNK


def _emb_body(ids_hbm, table_hbm, out_hbm, idx_v, rows_v, sem):
    wid = lax.axis_index("s") * NUM_CORES + lax.axis_index("c")
    base = wid * B_PER_W

    pltpu.sync_copy(ids_hbm.at[pl.ds(base, B_PER_W)], idx_v)

    def step(c, carry):
        off = pl.multiple_of(c * CHUNK, 8)
        pltpu.async_copy(
            table_hbm.at[idx_v.at[pl.ds(off, CHUNK)]], rows_v, sem).wait()
        pltpu.sync_copy(rows_v, out_hbm.at[pl.ds(base + off, CHUNK)])
        return carry

    lax.fori_loop(0, NCHUNK, step, None)


@jax.jit
def _emb(ids_flat, table):
    mesh = plsc.VectorSubcoreMesh(
        core_axis_name="c", subcore_axis_name="s",
        num_cores=NUM_CORES, num_subcores=NUM_SUBCORES)
    return pl.kernel(
        _emb_body,
        out_type=jax.ShapeDtypeStruct((TOT, D_MODEL), jnp.float32),
        mesh=mesh,
        scratch_types=[
            pltpu.VMEM((B_PER_W,), jnp.int32),
            pltpu.VMEM((CHUNK, D_MODEL), jnp.float32),
            pltpu.SemaphoreType.DMA,
        ],
    )(ids_flat, table)


def kernel(input_ids, table):
    ids_flat = input_ids.reshape(TOT).astype(jnp.int32)
    out = _emb(ids_flat, table)
    return out.reshape(BATCH, SEQ_LEN, D_MODEL)


# speedup vs baseline: 1.6807x; 1.1665x over previous
import jax
import jax.numpy as jnp
from jax import lax
from jax.experimental import pallas as pl
from jax.experimental.pallas import tpu as pltpu
from jax.experimental.pallas import tpu_sc as plsc

BATCH = 4
SEQ_LEN = 4096
D_MODEL = 1024
TOT = BATCH * SEQ_LEN

NUM_CORES = 2
NUM_SUBCORES = 16
NW = NUM_CORES * NUM_SUBCORES
B_PER_W = TOT // NW

CHUNK = 16
NBUF = 4
NCHUNK = B_PER_W // CHUNK
NGROUP = NCHUNK // NBUF


def _emb_body(ids_hbm, table_hbm, out_hbm, idx_v, rows_v, gsems, ssems):
    wid = lax.axis_index("s") * NUM_CORES + lax.axis_index("c")
    base = wid * B_PER_W

    pltpu.sync_copy(ids_hbm.at[pl.ds(base, B_PER_W)], idx_v)

    def gather_start(c, b):
        off = pl.multiple_of(c * CHUNK, 8)
        pltpu.async_copy(
            table_hbm.at[idx_v.at[pl.ds(off, CHUNK)]], rows_v.at[b],
            gsems.at[b])

    def gather_wait(b):
        pltpu.make_async_copy(
            table_hbm.at[idx_v.at[pl.ds(0, CHUNK)]], rows_v.at[b],
            gsems.at[b]).wait()

    def store_start(c, b):
        off = pl.multiple_of(c * CHUNK, 8)
        pltpu.async_copy(
            rows_v.at[b], out_hbm.at[pl.ds(base + off, CHUNK)], ssems.at[b])

    def store_wait(b):
        pltpu.make_async_copy(
            rows_v.at[b], out_hbm.at[pl.ds(base, CHUNK)], ssems.at[b]).wait()

    for b in range(NBUF):
        gather_start(b, b)

    def group(g, carry):
        for b in range(NBUF):
            c = g * NBUF + b
            gather_wait(b)
            store_start(c, b)
            store_wait(b)
            gather_start(c + NBUF, b)
        return carry

    lax.fori_loop(0, NGROUP - 1, group, None)

    for b in range(NBUF):
        c = NCHUNK - NBUF + b
        gather_wait(b)
        store_start(c, b)
    for b in range(NBUF):
        store_wait(b)


@jax.jit
def _emb(ids_flat, table):
    mesh = plsc.VectorSubcoreMesh(
        core_axis_name="c", subcore_axis_name="s",
        num_cores=NUM_CORES, num_subcores=NUM_SUBCORES)
    return pl.kernel(
        _emb_body,
        out_type=jax.ShapeDtypeStruct((TOT, D_MODEL), jnp.float32),
        mesh=mesh,
        scratch_types=[
            pltpu.VMEM((B_PER_W,), jnp.int32),
            pltpu.VMEM((NBUF, CHUNK, D_MODEL), jnp.float32),
            pltpu.SemaphoreType.DMA((NBUF,)),
            pltpu.SemaphoreType.DMA((NBUF,)),
        ],
    )(ids_flat, table)


def kernel(input_ids, table):
    ids_flat = input_ids.reshape(TOT).astype(jnp.int32)
    out = _emb(ids_flat, table)
    return out.reshape(BATCH, SEQ_LEN, D_MODEL)
